# baseline (speedup 1.0000x reference)
import jax
import jax.numpy as jnp
from jax.experimental import pallas as pl
from jax.experimental.pallas import tpu as pltpu

_B, _S, _E = 64, 1024, 1024
_T = 768
_NE = 16
_K = 2
_NC = 2
_BPC = _B // _NC


def _gate_kernel(x_ref, text_ref, w1_ref, b1_ref, w2_ref, b2_ref,
                 w_out_ref, i_out_ref, l_out_ref, acc_ref):
    j = pl.program_id(1)
    s = jnp.sum(x_ref[0], axis=0, keepdims=True)
    acc_ref[pl.ds(j, 1), :] = s

    @pl.when(j == _BPC - 1)
    def _epilogue():
        mean = acc_ref[...] * (1.0 / _S)
        text = text_ref[...]
        w1a = w1_ref[0:_E, :]
        w1b = w1_ref[_E:_E + _T, :]
        h = jnp.dot(mean, w1a, preferred_element_type=jnp.float32)
        h = h + jnp.dot(text, w1b, preferred_element_type=jnp.float32)
        h = jnp.maximum(h + b1_ref[...], 0.0)
        logits = (jnp.dot(h, w2_ref[...], preferred_element_type=jnp.float32)
                  + b2_ref[...])
        l_out_ref[...] = logits

        lane = jax.lax.broadcasted_iota(jnp.int32, (_BPC, _NE), 1)
        m1 = jnp.max(logits, axis=1, keepdims=True)
        i1 = jnp.min(jnp.where(logits == m1, lane, _NE), axis=1, keepdims=True)
        masked = jnp.where(lane == i1, -jnp.inf, logits)
        m2 = jnp.max(masked, axis=1, keepdims=True)
        i2 = jnp.min(jnp.where(masked == m2, lane, _NE), axis=1, keepdims=True)

        lane2 = jax.lax.broadcasted_iota(jnp.int32, (_BPC, _K), 1)
        i_out_ref[...] = jnp.where(lane2 == 0, i1, i2)
        e2 = jnp.exp(m2 - m1)
        denom = 1.0 + e2
        w_out_ref[...] = jnp.where(lane2 == 0, 1.0 / denom, e2 / denom)


def kernel(x, text_embedding, W1, b1, W2, b2):
    b1r = b1.reshape(1, _E)
    b2r = b2.reshape(1, _NE)
    out_shape = (
        jax.ShapeDtypeStruct((_B, _K), jnp.float32),
        jax.ShapeDtypeStruct((_B, _K), jnp.int32),
        jax.ShapeDtypeStruct((_B, _NE), jnp.float32),
    )
    grid = (_NC, _BPC)
    weights, indices, logits = pl.pallas_call(
        _gate_kernel,
        grid=grid,
        in_specs=[
            pl.BlockSpec((1, _S, _E), lambda c, j: (c * _BPC + j, 0, 0)),
            pl.BlockSpec((_BPC, _T), lambda c, j: (c, 0)),
            pl.BlockSpec((_E + _T, _E), lambda c, j: (0, 0)),
            pl.BlockSpec((1, _E), lambda c, j: (0, 0)),
            pl.BlockSpec((_E, _NE), lambda c, j: (0, 0)),
            pl.BlockSpec((1, _NE), lambda c, j: (0, 0)),
        ],
        out_specs=(
            pl.BlockSpec((_BPC, _K), lambda c, j: (c, 0)),
            pl.BlockSpec((_BPC, _K), lambda c, j: (c, 0)),
            pl.BlockSpec((_BPC, _NE), lambda c, j: (c, 0)),
        ),
        out_shape=out_shape,
        scratch_shapes=[pltpu.VMEM((_BPC, _E), jnp.float32)],
        compiler_params=pltpu.CompilerParams(
            dimension_semantics=("parallel", "arbitrary"),
        ),
    )(x, text_embedding, W1, b1r, W2, b2r)
    return (weights, indices, logits)


# speedup vs baseline: 1.0790x; 1.0790x over previous
import jax
import jax.numpy as jnp
from jax.experimental import pallas as pl
from jax.experimental.pallas import tpu as pltpu

_B, _S, _E = 64, 1024, 1024
_T = 768
_NE = 16
_K = 2
_NBUF = 4


def _gate_kernel(x_hbm, text_ref, w1_hbm, b1_ref, w2_ref, b2_ref,
                 w_out_ref, i_out_ref, l_out_ref,
                 buf, w1_v, acc_ref, sems, w1_sem):
    pltpu.make_async_copy(w1_hbm, w1_v, w1_sem).start()
    for r in range(_NBUF):
        pltpu.make_async_copy(x_hbm.at[r], buf.at[r], sems.at[r]).start()

    def outer(o, carry):
        for r in range(_NBUF):
            b = o * _NBUF + r
            pltpu.make_async_copy(x_hbm.at[b], buf.at[r], sems.at[r]).wait()
            s = jnp.sum(buf[r], axis=0, keepdims=True)
            acc_ref[pl.ds(b, 1), :] = s
            nb = b + _NBUF

            @pl.when(nb < _B)
            def _():
                pltpu.make_async_copy(x_hbm.at[nb], buf.at[r], sems.at[r]).start()
        return carry

    jax.lax.fori_loop(0, _B // _NBUF, outer, 0)

    pltpu.make_async_copy(w1_hbm, w1_v, w1_sem).wait()
    mean = acc_ref[...] * (1.0 / _S)
    text = text_ref[...]
    w1a = w1_v[0:_E, :]
    w1b = w1_v[_E:_E + _T, :]
    h = jnp.dot(mean, w1a, preferred_element_type=jnp.float32)
    h = h + jnp.dot(text, w1b, preferred_element_type=jnp.float32)
    h = jnp.maximum(h + b1_ref[...], 0.0)
    logits = (jnp.dot(h, w2_ref[...], preferred_element_type=jnp.float32)
              + b2_ref[...])
    l_out_ref[...] = logits

    lane = jax.lax.broadcasted_iota(jnp.int32, (_B, _NE), 1)
    m1 = jnp.max(logits, axis=1, keepdims=True)
    i1 = jnp.min(jnp.where(logits == m1, lane, _NE), axis=1, keepdims=True)
    masked = jnp.where(lane == i1, -jnp.inf, logits)
    m2 = jnp.max(masked, axis=1, keepdims=True)
    i2 = jnp.min(jnp.where(masked == m2, lane, _NE), axis=1, keepdims=True)

    lane2 = jax.lax.broadcasted_iota(jnp.int32, (_B, _K), 1)
    i_out_ref[...] = jnp.where(lane2 == 0, i1, i2)
    e2 = jnp.exp(m2 - m1)
    denom = 1.0 + e2
    w_out_ref[...] = jnp.where(lane2 == 0, 1.0 / denom, e2 / denom)


def kernel(x, text_embedding, W1, b1, W2, b2):
    b1r = b1.reshape(1, _E)
    b2r = b2.reshape(1, _NE)
    out_shape = (
        jax.ShapeDtypeStruct((_B, _K), jnp.float32),
        jax.ShapeDtypeStruct((_B, _K), jnp.int32),
        jax.ShapeDtypeStruct((_B, _NE), jnp.float32),
    )
    weights, indices, logits = pl.pallas_call(
        _gate_kernel,
        in_specs=[
            pl.BlockSpec(memory_space=pl.ANY),
            pl.BlockSpec(memory_space=pltpu.MemorySpace.VMEM),
            pl.BlockSpec(memory_space=pl.ANY),
            pl.BlockSpec(memory_space=pltpu.MemorySpace.VMEM),
            pl.BlockSpec(memory_space=pltpu.MemorySpace.VMEM),
            pl.BlockSpec(memory_space=pltpu.MemorySpace.VMEM),
        ],
        out_specs=(
            pl.BlockSpec(memory_space=pltpu.MemorySpace.VMEM),
            pl.BlockSpec(memory_space=pltpu.MemorySpace.VMEM),
            pl.BlockSpec(memory_space=pltpu.MemorySpace.VMEM),
        ),
        out_shape=out_shape,
        scratch_shapes=[
            pltpu.VMEM((_NBUF, _S, _E), jnp.float32),
            pltpu.VMEM((_E + _T, _E), jnp.float32),
            pltpu.VMEM((_B, _E), jnp.float32),
            pltpu.SemaphoreType.DMA((_NBUF,)),
            pltpu.SemaphoreType.DMA,
        ],
    )(x, text_embedding, W1, b1r, W2, b2r)
    return (weights, indices, logits)
